# baseline (speedup 1.0000x reference)
import functools

import jax
import jax.numpy as jnp
from jax import lax
from jax.experimental import pallas as pl
from jax.experimental.pallas import tpu as pltpu
from jax.experimental.pallas import tpu_sc as plsc

N = 10000
D = 128
NG = 64
E = 320000

NPAD = 10112
ROWS_PER_TILE = 640
ACC_ROWS = 10240
CHUNK = 128
NC = 2
NS = 16
NW = NC * NS
CHUNKS_PER_W = 80
E_PAD = NW * CHUNKS_PER_W * CHUNK

def _zero_vmem_2d(ref, rows, cols):
    def body(r, _):
        for k in range(cols // 16):
            ref[r, pl.ds(k * 16, 16)] = jnp.zeros((16,), jnp.float32)
        return 0
    lax.fori_loop(0, rows, body, 0)


def _zero_vmem_1d(ref, n):
    def body(j, _):
        ref[pl.ds(j * 16, 16)] = jnp.zeros((16,), jnp.float32)
        return 0
    lax.fori_loop(0, n // 16, body, 0)


@functools.cache
def _make_sc_degree():
    return functools.partial(
        pl.kernel,
        out_type=jax.ShapeDtypeStruct((NC, ACC_ROWS), jnp.float32),
        mesh=plsc.VectorSubcoreMesh(core_axis_name="c", subcore_axis_name="s"),
        scratch_types=[
            pltpu.VMEM((CHUNKS_PER_W, CHUNK), jnp.int32),
            pltpu.VMEM((CHUNK,), jnp.float32),
            pltpu.VMEM((ROWS_PER_TILE,), jnp.float32),
            pltpu.VMEM_SHARED((ACC_ROWS,), jnp.float32),
            pltpu.SemaphoreType.DMA,
        ],
    )(_sc_degree_body)


def _sc_degree_body(idx_h, out_h, didx_all, ones_v, zbuf, acc, sem):
    c = lax.axis_index("c")
    s = lax.axis_index("s")
    wid = s * NC + c

    def setone(j, _):
        ones_v[pl.ds(j * 16, 16)] = jnp.ones((16,), jnp.float32)
        return 0
    lax.fori_loop(0, CHUNK // 16, setone, 0)
    _zero_vmem_1d(zbuf, ROWS_PER_TILE)
    pltpu.sync_copy(idx_h.at[1, wid], didx_all)
    pltpu.sync_copy(zbuf, acc.at[pl.ds(s * ROWS_PER_TILE, ROWS_PER_TILE)])
    plsc.subcore_barrier()

    W = 8

    def fire(j, _):
        pltpu.async_copy(ones_v, acc.at[didx_all.at[j]], sem, add=True)

        @pl.when(j >= W)
        def _():
            pltpu.make_async_copy(
                ones_v, acc.at[didx_all.at[j - W]], sem).wait()
        return 0
    lax.fori_loop(0, CHUNKS_PER_W, fire, 0)

    def drain(j, _):
        pltpu.make_async_copy(
            ones_v, acc.at[didx_all.at[CHUNKS_PER_W - W + j]], sem).wait()
        return 0
    lax.fori_loop(0, W, drain, 0)
    plsc.subcore_barrier()
    pltpu.sync_copy(acc.at[pl.ds(s * ROWS_PER_TILE, ROWS_PER_TILE)],
                    out_h.at[c, pl.ds(s * ROWS_PER_TILE, ROWS_PER_TILE)])


@functools.cache
def _make_sc_scatter():
    return functools.partial(
        pl.kernel,
        out_type=jax.ShapeDtypeStruct((NC, ACC_ROWS, D), jnp.float32),
        mesh=plsc.VectorSubcoreMesh(core_axis_name="c", subcore_axis_name="s"),
        scratch_types=[
            pltpu.VMEM((CHUNKS_PER_W // 2, CHUNK), jnp.int32),
            pltpu.VMEM((CHUNKS_PER_W // 2, CHUNK), jnp.int32),
            pltpu.VMEM((CHUNK, D), jnp.float32),
            pltpu.VMEM((CHUNK, D), jnp.float32),
            pltpu.VMEM_SHARED((ACC_ROWS, D), jnp.float32),
            pltpu.SemaphoreType.DMA,
            pltpu.SemaphoreType.DMA,
            pltpu.SemaphoreType.DMA,
            pltpu.SemaphoreType.DMA,
        ],
    )(_sc_scatter_body)


def _sc_scatter_body(table_h, idx_h, out_h, sidx, didx, rows0, rows1,
                     acc, g0, g1, s0, s1):
    c = lax.axis_index("c")
    s = lax.axis_index("s")
    wid = s * NC + c
    half = CHUNKS_PER_W // 2
    nt = half // 2
    pltpu.sync_copy(idx_h.at[0, wid, pl.ds(0, half)], sidx)
    pltpu.sync_copy(idx_h.at[1, wid, pl.ds(0, half)], didx)
    pltpu.async_copy(table_h.at[sidx.at[0]], rows0, g0)
    _zero_vmem_2d(rows1, CHUNK, D)

    def zeroacc(k, _):
        pltpu.sync_copy(
            rows1, acc.at[pl.ds(s * ROWS_PER_TILE + k * CHUNK, CHUNK)])
        return 0
    lax.fori_loop(0, ROWS_PER_TILE // CHUNK, zeroacc, 0)
    plsc.subcore_barrier()

    for phase in range(2):
        if phase:
            pltpu.sync_copy(idx_h.at[0, wid, pl.ds(phase * half, half)], sidx)
            pltpu.sync_copy(idx_h.at[1, wid, pl.ds(phase * half, half)], didx)
            pltpu.async_copy(table_h.at[sidx.at[0]], rows0, g0)

        def body(t, _):
            j0 = 2 * t
            pltpu.make_async_copy(table_h.at[sidx.at[j0]], rows0, g0).wait()
            pltpu.async_copy(rows0, acc.at[didx.at[j0]], s0, add=True)

            @pl.when(t > 0)
            def _():
                pltpu.make_async_copy(
                    rows1, acc.at[didx.at[j0 - 1]], s1).wait()
            pltpu.async_copy(table_h.at[sidx.at[j0 + 1]], rows1, g1)
            pltpu.make_async_copy(table_h.at[sidx.at[j0 + 1]], rows1, g1).wait()
            pltpu.async_copy(rows1, acc.at[didx.at[j0 + 1]], s1, add=True)
            pltpu.make_async_copy(rows0, acc.at[didx.at[j0]], s0).wait()

            @pl.when(t < nt - 1)
            def _():
                pltpu.async_copy(table_h.at[sidx.at[j0 + 2]], rows0, g0)
            return 0
        lax.fori_loop(0, nt, body, 0)
        pltpu.make_async_copy(rows1, acc.at[didx.at[half - 1]], s1).wait()
    plsc.subcore_barrier()
    pltpu.sync_copy(acc.at[pl.ds(s * ROWS_PER_TILE, ROWS_PER_TILE)],
                    out_h.at[c, pl.ds(s * ROWS_PER_TILE, ROWS_PER_TILE)])


_BM = 632


def _project0(x_pad, W1, degp3):
    def body(x_ref, w_ref, d0_ref, d1_ref, h_ref, dinv_ref):
        i = pl.program_id(0)
        deg = d0_ref[0] + d1_ref[0] + 1.0
        row = lax.broadcasted_iota(jnp.int32, (_BM, 1), 0) + i * _BM
        dinv = jnp.where(row < N, lax.rsqrt(deg), 0.0)
        h = jnp.dot(x_ref[...], w_ref[...], preferred_element_type=jnp.float32)
        h_ref[...] = h * dinv
        dinv_ref[...] = dinv
    return pl.pallas_call(
        body,
        grid=(NPAD // _BM,),
        in_specs=[
            pl.BlockSpec((_BM, D), lambda i: (i, 0)),
            pl.BlockSpec((D, D), lambda i: (0, 0)),
            pl.BlockSpec((1, _BM, 1), lambda i: (0, i, 0)),
            pl.BlockSpec((1, _BM, 1), lambda i: (1, i, 0)),
        ],
        out_specs=[
            pl.BlockSpec((_BM, D), lambda i: (i, 0)),
            pl.BlockSpec((_BM, 1), lambda i: (i, 0)),
        ],
        out_shape=[
            jax.ShapeDtypeStruct((NPAD, D), jnp.float32),
            jax.ShapeDtypeStruct((NPAD, 1), jnp.float32),
        ],
    )(x_pad, W1, degp3, degp3)


def _combine_project(accp, hprev, dinv, b, Wn):
    def body(p0_ref, p1_ref, h_ref, dinv_ref, b_ref, w_ref, o_ref):
        a = p0_ref[0] + p1_ref[0]
        z = dinv_ref[...] * (a + h_ref[...]) + b_ref[...]
        y = jnp.maximum(z, 0.0)
        o_ref[...] = jnp.dot(
            y, w_ref[...], preferred_element_type=jnp.float32) * dinv_ref[...]
    return pl.pallas_call(
        body,
        grid=(NPAD // _BM,),
        in_specs=[
            pl.BlockSpec((1, _BM, D), lambda i: (0, i, 0)),
            pl.BlockSpec((1, _BM, D), lambda i: (1, i, 0)),
            pl.BlockSpec((_BM, D), lambda i: (i, 0)),
            pl.BlockSpec((_BM, 1), lambda i: (i, 0)),
            pl.BlockSpec((1, D), lambda i: (0, 0)),
            pl.BlockSpec((D, D), lambda i: (0, 0)),
        ],
        out_specs=pl.BlockSpec((_BM, D), lambda i: (i, 0)),
        out_shape=jax.ShapeDtypeStruct((NPAD, D), jnp.float32),
    )(accp, accp, hprev, dinv, b.reshape(1, D), Wn)


def _combine_pool_fc(accp, hprev, dinv, b, batch2, fcW, fcb):
    nsteps = NPAD // CHUNK

    def body(p0_ref, p1_ref, h_ref, dinv_ref, b_ref, bt_ref, w_ref, fb_ref,
             o_ref, pool_s, cnt_s):
        i = pl.program_id(0)

        @pl.when(i == 0)
        def _():
            pool_s[...] = jnp.zeros_like(pool_s)
            cnt_s[...] = jnp.zeros_like(cnt_s)

        a = p0_ref[0] + p1_ref[0]
        z = dinv_ref[...] * (a + h_ref[...]) + b_ref[...]
        y = jnp.maximum(z, 0.0)
        gids = lax.broadcasted_iota(jnp.int32, (NG, CHUNK), 0)
        maskf = jnp.where(gids == bt_ref[0], 1.0, 0.0)
        pool_s[...] += jnp.dot(maskf, y, preferred_element_type=jnp.float32)
        cnt_s[...] += jnp.sum(maskf, axis=1, keepdims=True)

        @pl.when(i == nsteps - 1)
        def _():
            g = pool_s[...] / jnp.maximum(cnt_s[...], 1.0)
            o = jnp.dot(g, w_ref[...], preferred_element_type=jnp.float32)
            o_ref[...] = jax.nn.sigmoid(o + fb_ref[...])

    return pl.pallas_call(
        body,
        grid=(nsteps,),
        in_specs=[
            pl.BlockSpec((1, CHUNK, D), lambda i: (0, i, 0)),
            pl.BlockSpec((1, CHUNK, D), lambda i: (1, i, 0)),
            pl.BlockSpec((CHUNK, D), lambda i: (i, 0)),
            pl.BlockSpec((CHUNK, 1), lambda i: (i, 0)),
            pl.BlockSpec((1, D), lambda i: (0, 0)),
            pl.BlockSpec((1, 1, CHUNK), lambda i: (i, 0, 0)),
            pl.BlockSpec((D, 1), lambda i: (0, 0)),
            pl.BlockSpec((1, 1), lambda i: (0, 0)),
        ],
        out_specs=pl.BlockSpec((NG, 1), lambda i: (0, 0)),
        out_shape=jax.ShapeDtypeStruct((NG, 1), jnp.float32),
        scratch_shapes=[
            pltpu.VMEM((NG, D), jnp.float32),
            pltpu.VMEM((NG, 1), jnp.float32),
        ],
    )(accp, accp, hprev, dinv, b.reshape(1, D), batch2, fcW,
      fcb.reshape(1, 1))


def kernel(x, edge_index, batch, W1, b1, W2, b2, W3, b3, fcW, fcb):
    ei = edge_index.astype(jnp.int32)
    pad_idx = N + (jnp.arange(E_PAD - E, dtype=jnp.int32) % (NPAD - N))
    pad2 = jnp.broadcast_to(pad_idx, (2, E_PAD - E))
    idx4 = jnp.concatenate([ei, pad2], axis=1).reshape(
        2, NW, CHUNKS_PER_W, CHUNK)

    degp = _make_sc_degree()(idx4)

    x_pad = jnp.pad(x, ((0, NPAD - N), (0, 0)))
    h1, dinv = _project0(x_pad, W1, degp.reshape(NC, ACC_ROWS, 1))

    acc1 = _make_sc_scatter()(h1, idx4)
    h2 = _combine_project(acc1, h1, dinv, b1, W2)
    acc2 = _make_sc_scatter()(h2, idx4)
    h3 = _combine_project(acc2, h2, dinv, b2, W3)
    acc3 = _make_sc_scatter()(h3, idx4)

    batch2 = jnp.pad(batch.astype(jnp.int32), (0, NPAD - N),
                     constant_values=NG).reshape(NPAD // CHUNK, 1, CHUNK)
    return _combine_pool_fc(acc3, h3, dinv, b3, batch2, fcW, fcb)


# speedup vs baseline: 1.0278x; 1.0278x over previous
import functools

import jax
import jax.numpy as jnp
from jax import lax
from jax.experimental import pallas as pl
from jax.experimental.pallas import tpu as pltpu
from jax.experimental.pallas import tpu_sc as plsc

N = 10000
D = 128
NG = 64
E = 320000

NPAD = 10112
ROWS_PER_TILE = 640
ACC_ROWS = 10240
CHUNK = 128
NC = 2
NS = 16
NW = NC * NS
CHUNKS_PER_W = 80
E_PAD = NW * CHUNKS_PER_W * CHUNK

def _zero_vmem_2d(ref, rows, cols):
    def body(r, _):
        for k in range(cols // 16):
            ref[r, pl.ds(k * 16, 16)] = jnp.zeros((16,), jnp.float32)
        return 0
    lax.fori_loop(0, rows, body, 0)


def _zero_vmem_1d(ref, n):
    def body(j, _):
        ref[pl.ds(j * 16, 16)] = jnp.zeros((16,), jnp.float32)
        return 0
    lax.fori_loop(0, n // 16, body, 0)


@functools.cache
def _make_sc_degree():
    return functools.partial(
        pl.kernel,
        out_type=jax.ShapeDtypeStruct((NC, ACC_ROWS), jnp.float32),
        mesh=plsc.VectorSubcoreMesh(core_axis_name="c", subcore_axis_name="s"),
        scratch_types=[
            pltpu.VMEM((CHUNKS_PER_W, CHUNK), jnp.int32),
            pltpu.VMEM((CHUNK,), jnp.float32),
            pltpu.VMEM((ROWS_PER_TILE,), jnp.float32),
            pltpu.VMEM_SHARED((ACC_ROWS,), jnp.float32),
            pltpu.SemaphoreType.DMA,
        ],
    )(_sc_degree_body)


def _sc_degree_body(idx_h, out_h, didx_all, ones_v, zbuf, acc, sem):
    c = lax.axis_index("c")
    s = lax.axis_index("s")
    wid = s * NC + c

    def setone(j, _):
        ones_v[pl.ds(j * 16, 16)] = jnp.ones((16,), jnp.float32)
        return 0
    lax.fori_loop(0, CHUNK // 16, setone, 0)
    _zero_vmem_1d(zbuf, ROWS_PER_TILE)
    pltpu.sync_copy(idx_h.at[1, wid], didx_all)
    pltpu.sync_copy(zbuf, acc.at[pl.ds(s * ROWS_PER_TILE, ROWS_PER_TILE)])
    plsc.subcore_barrier()

    W = 8

    def fire(j, _):
        pltpu.async_copy(ones_v, acc.at[didx_all.at[j]], sem, add=True)

        @pl.when(j >= W)
        def _():
            pltpu.make_async_copy(
                ones_v, acc.at[didx_all.at[j - W]], sem).wait()
        return 0
    lax.fori_loop(0, CHUNKS_PER_W, fire, 0)

    def drain(j, _):
        pltpu.make_async_copy(
            ones_v, acc.at[didx_all.at[CHUNKS_PER_W - W + j]], sem).wait()
        return 0
    lax.fori_loop(0, W, drain, 0)
    plsc.subcore_barrier()
    pltpu.sync_copy(acc.at[pl.ds(s * ROWS_PER_TILE, ROWS_PER_TILE)],
                    out_h.at[c, pl.ds(s * ROWS_PER_TILE, ROWS_PER_TILE)])


@functools.cache
def _make_sc_scatter():
    return functools.partial(
        pl.kernel,
        out_type=jax.ShapeDtypeStruct((NC, ACC_ROWS, D), jnp.float32),
        mesh=plsc.VectorSubcoreMesh(core_axis_name="c", subcore_axis_name="s"),
        scratch_types=[
            pltpu.VMEM((CHUNKS_PER_W // 2, CHUNK), jnp.int32),
            pltpu.VMEM((CHUNKS_PER_W // 2, CHUNK), jnp.int32),
            pltpu.VMEM((CHUNK, D), jnp.float32),
            pltpu.VMEM((CHUNK, D), jnp.float32),
            pltpu.VMEM_SHARED((ACC_ROWS, D), jnp.float32),
            pltpu.SemaphoreType.DMA,
            pltpu.SemaphoreType.DMA,
            pltpu.SemaphoreType.DMA,
            pltpu.SemaphoreType.DMA,
        ],
    )(_sc_scatter_body)


def _sc_scatter_body(table_h, idx_h, out_h, sidx, didx, rows0, rows1,
                     acc, g0, g1, s0, s1):
    c = lax.axis_index("c")
    s = lax.axis_index("s")
    wid = s * NC + c
    half = CHUNKS_PER_W // 2
    nt = half // 2
    pltpu.sync_copy(idx_h.at[0, wid, pl.ds(0, half)], sidx)
    pltpu.sync_copy(idx_h.at[1, wid, pl.ds(0, half)], didx)
    pltpu.async_copy(table_h.at[sidx.at[0]], rows0, g0)
    _zero_vmem_2d(rows1, CHUNK, D)

    def zeroacc(k, _):
        pltpu.sync_copy(
            rows1, acc.at[pl.ds(s * ROWS_PER_TILE + k * CHUNK, CHUNK)])
        return 0
    lax.fori_loop(0, ROWS_PER_TILE // CHUNK, zeroacc, 0)
    plsc.subcore_barrier()

    for phase in range(2):
        if phase:
            pltpu.sync_copy(idx_h.at[0, wid, pl.ds(phase * half, half)], sidx)
            pltpu.sync_copy(idx_h.at[1, wid, pl.ds(phase * half, half)], didx)
            pltpu.async_copy(table_h.at[sidx.at[0]], rows0, g0)

        def body(t, _):
            j0 = 2 * t
            pltpu.make_async_copy(table_h.at[sidx.at[j0]], rows0, g0).wait()
            pltpu.async_copy(rows0, acc.at[didx.at[j0]], s0, add=True)

            @pl.when(t > 0)
            def _():
                pltpu.make_async_copy(
                    rows1, acc.at[didx.at[j0 - 1]], s1).wait()
            pltpu.async_copy(table_h.at[sidx.at[j0 + 1]], rows1, g1)
            pltpu.make_async_copy(table_h.at[sidx.at[j0 + 1]], rows1, g1).wait()
            pltpu.async_copy(rows1, acc.at[didx.at[j0 + 1]], s1, add=True)
            pltpu.make_async_copy(rows0, acc.at[didx.at[j0]], s0).wait()

            @pl.when(t < nt - 1)
            def _():
                pltpu.async_copy(table_h.at[sidx.at[j0 + 2]], rows0, g0)
            return 0
        lax.fori_loop(0, nt, body, 0)
        pltpu.make_async_copy(rows1, acc.at[didx.at[half - 1]], s1).wait()
    plsc.subcore_barrier()
    pltpu.sync_copy(acc.at[pl.ds(s * ROWS_PER_TILE, ROWS_PER_TILE)],
                    out_h.at[c, pl.ds(s * ROWS_PER_TILE, ROWS_PER_TILE)])


_BM = 1264


def _project0(x, W1, degp3):
    def body(x_ref, w_ref, d0_ref, d1_ref, h_ref, dinv_ref):
        i = pl.program_id(0)
        deg = d0_ref[0] + d1_ref[0] + 1.0
        row = lax.broadcasted_iota(jnp.int32, (_BM, 1), 0) + i * _BM
        valid = row < N
        dinv = jnp.where(valid, lax.rsqrt(deg), 0.0)
        h = jnp.dot(x_ref[...], w_ref[...], preferred_element_type=jnp.float32)
        h_ref[...] = jnp.where(valid, h * dinv, 0.0)
        dinv_ref[...] = dinv
    return pl.pallas_call(
        body,
        grid=(NPAD // _BM,),
        in_specs=[
            pl.BlockSpec((_BM, D), lambda i: (i, 0)),
            pl.BlockSpec((D, D), lambda i: (0, 0)),
            pl.BlockSpec((1, _BM, 1), lambda i: (0, i, 0)),
            pl.BlockSpec((1, _BM, 1), lambda i: (1, i, 0)),
        ],
        out_specs=[
            pl.BlockSpec((_BM, D), lambda i: (i, 0)),
            pl.BlockSpec((_BM, 1), lambda i: (i, 0)),
        ],
        out_shape=[
            jax.ShapeDtypeStruct((NPAD, D), jnp.float32),
            jax.ShapeDtypeStruct((NPAD, 1), jnp.float32),
        ],
    )(x, W1, degp3, degp3)


def _combine_project(accp, hprev, dinv, b, Wn):
    def body(p0_ref, p1_ref, h_ref, dinv_ref, b_ref, w_ref, o_ref):
        a = p0_ref[0] + p1_ref[0]
        z = dinv_ref[...] * (a + h_ref[...]) + b_ref[...]
        y = jnp.maximum(z, 0.0)
        o_ref[...] = jnp.dot(
            y, w_ref[...], preferred_element_type=jnp.float32) * dinv_ref[...]
    return pl.pallas_call(
        body,
        grid=(NPAD // _BM,),
        in_specs=[
            pl.BlockSpec((1, _BM, D), lambda i: (0, i, 0)),
            pl.BlockSpec((1, _BM, D), lambda i: (1, i, 0)),
            pl.BlockSpec((_BM, D), lambda i: (i, 0)),
            pl.BlockSpec((_BM, 1), lambda i: (i, 0)),
            pl.BlockSpec((1, D), lambda i: (0, 0)),
            pl.BlockSpec((D, D), lambda i: (0, 0)),
        ],
        out_specs=pl.BlockSpec((_BM, D), lambda i: (i, 0)),
        out_shape=jax.ShapeDtypeStruct((NPAD, D), jnp.float32),
    )(accp, accp, hprev, dinv, b.reshape(1, D), Wn)


def _combine_pool_fc(accp, hprev, dinv, b, batch2, fcW, fcb):
    nsteps = NPAD // CHUNK

    def body(p0_ref, p1_ref, h_ref, dinv_ref, b_ref, bt_ref, w_ref, fb_ref,
             o_ref, pool_s, cnt_s):
        i = pl.program_id(0)

        @pl.when(i == 0)
        def _():
            pool_s[...] = jnp.zeros_like(pool_s)
            cnt_s[...] = jnp.zeros_like(cnt_s)

        a = p0_ref[0] + p1_ref[0]
        z = dinv_ref[...] * (a + h_ref[...]) + b_ref[...]
        y = jnp.maximum(z, 0.0)
        gids = lax.broadcasted_iota(jnp.int32, (NG, CHUNK), 0)
        maskf = jnp.where(gids == bt_ref[0], 1.0, 0.0)
        pool_s[...] += jnp.dot(maskf, y, preferred_element_type=jnp.float32)
        cnt_s[...] += jnp.sum(maskf, axis=1, keepdims=True)

        @pl.when(i == nsteps - 1)
        def _():
            g = pool_s[...] / jnp.maximum(cnt_s[...], 1.0)
            o = jnp.dot(g, w_ref[...], preferred_element_type=jnp.float32)
            o_ref[...] = jax.nn.sigmoid(o + fb_ref[...])

    return pl.pallas_call(
        body,
        grid=(nsteps,),
        in_specs=[
            pl.BlockSpec((1, CHUNK, D), lambda i: (0, i, 0)),
            pl.BlockSpec((1, CHUNK, D), lambda i: (1, i, 0)),
            pl.BlockSpec((CHUNK, D), lambda i: (i, 0)),
            pl.BlockSpec((CHUNK, 1), lambda i: (i, 0)),
            pl.BlockSpec((1, D), lambda i: (0, 0)),
            pl.BlockSpec((1, 1, CHUNK), lambda i: (i, 0, 0)),
            pl.BlockSpec((D, 1), lambda i: (0, 0)),
            pl.BlockSpec((1, 1), lambda i: (0, 0)),
        ],
        out_specs=pl.BlockSpec((NG, 1), lambda i: (0, 0)),
        out_shape=jax.ShapeDtypeStruct((NG, 1), jnp.float32),
        scratch_shapes=[
            pltpu.VMEM((NG, D), jnp.float32),
            pltpu.VMEM((NG, 1), jnp.float32),
        ],
    )(accp, accp, hprev, dinv, b.reshape(1, D), batch2, fcW,
      fcb.reshape(1, 1))


def kernel(x, edge_index, batch, W1, b1, W2, b2, W3, b3, fcW, fcb):
    ei = edge_index.astype(jnp.int32)
    pad_idx = N + (jnp.arange(E_PAD - E, dtype=jnp.int32) % (NPAD - N))
    pad2 = jnp.broadcast_to(pad_idx, (2, E_PAD - E))
    idx4 = jnp.concatenate([ei, pad2], axis=1).reshape(
        2, NW, CHUNKS_PER_W, CHUNK)

    degp = _make_sc_degree()(idx4)

    h1, dinv = _project0(x, W1, degp.reshape(NC, ACC_ROWS, 1))

    acc1 = _make_sc_scatter()(h1, idx4)
    h2 = _combine_project(acc1, h1, dinv, b1, W2)
    acc2 = _make_sc_scatter()(h2, idx4)
    h3 = _combine_project(acc2, h2, dinv, b2, W3)
    acc3 = _make_sc_scatter()(h3, idx4)

    batch2 = jnp.pad(batch.astype(jnp.int32), (0, NPAD - N),
                     constant_values=NG).reshape(NPAD // CHUNK, 1, CHUNK)
    return _combine_pool_fc(acc3, h3, dinv, b3, batch2, fcW, fcb)


# speedup vs baseline: 1.0309x; 1.0030x over previous
import functools

import jax
import jax.numpy as jnp
from jax import lax
from jax.experimental import pallas as pl
from jax.experimental.pallas import tpu as pltpu
from jax.experimental.pallas import tpu_sc as plsc

N = 10000
D = 128
NG = 64
E = 320000

NPAD = 10112
ROWS_PER_TILE = 640
ACC_ROWS = 10240
CHUNK = 128
NC = 2
NS = 16
NW = NC * NS
CHUNKS_PER_W = 80
E_PAD = NW * CHUNKS_PER_W * CHUNK

def _zero_vmem_2d(ref, rows, cols):
    def body(r, _):
        for k in range(cols // 16):
            ref[r, pl.ds(k * 16, 16)] = jnp.zeros((16,), jnp.float32)
        return 0
    lax.fori_loop(0, rows, body, 0)


def _zero_vmem_1d(ref, n):
    def body(j, _):
        ref[pl.ds(j * 16, 16)] = jnp.zeros((16,), jnp.float32)
        return 0
    lax.fori_loop(0, n // 16, body, 0)


@functools.cache
def _make_sc_degree():
    return functools.partial(
        pl.kernel,
        out_type=jax.ShapeDtypeStruct((NC, ACC_ROWS), jnp.float32),
        mesh=plsc.VectorSubcoreMesh(core_axis_name="c", subcore_axis_name="s"),
        scratch_types=[
            pltpu.VMEM((CHUNKS_PER_W, CHUNK), jnp.int32),
            pltpu.VMEM((CHUNK,), jnp.float32),
            pltpu.VMEM((ROWS_PER_TILE,), jnp.float32),
            pltpu.VMEM_SHARED((ACC_ROWS,), jnp.float32),
            pltpu.SemaphoreType.DMA,
        ],
    )(_sc_degree_body)


def _sc_degree_body(idx_h, out_h, didx_all, ones_v, zbuf, acc, sem):
    c = lax.axis_index("c")
    s = lax.axis_index("s")
    wid = s * NC + c

    def setone(j, _):
        ones_v[pl.ds(j * 16, 16)] = jnp.ones((16,), jnp.float32)
        return 0
    lax.fori_loop(0, CHUNK // 16, setone, 0)
    _zero_vmem_1d(zbuf, ROWS_PER_TILE)
    pltpu.sync_copy(idx_h.at[1, wid], didx_all)
    pltpu.sync_copy(zbuf, acc.at[pl.ds(s * ROWS_PER_TILE, ROWS_PER_TILE)])
    plsc.subcore_barrier()

    W = 8

    def fire(j, _):
        pltpu.async_copy(ones_v, acc.at[didx_all.at[j]], sem, add=True)

        @pl.when(j >= W)
        def _():
            pltpu.make_async_copy(
                ones_v, acc.at[didx_all.at[j - W]], sem).wait()
        return 0
    lax.fori_loop(0, CHUNKS_PER_W, fire, 0)

    def drain(j, _):
        pltpu.make_async_copy(
            ones_v, acc.at[didx_all.at[CHUNKS_PER_W - W + j]], sem).wait()
        return 0
    lax.fori_loop(0, W, drain, 0)
    plsc.subcore_barrier()
    pltpu.sync_copy(acc.at[pl.ds(s * ROWS_PER_TILE, ROWS_PER_TILE)],
                    out_h.at[c, pl.ds(s * ROWS_PER_TILE, ROWS_PER_TILE)])


@functools.cache
def _make_sc_scatter():
    return functools.partial(
        pl.kernel,
        out_type=jax.ShapeDtypeStruct((NC, ACC_ROWS, D), jnp.float32),
        mesh=plsc.VectorSubcoreMesh(core_axis_name="c", subcore_axis_name="s"),
        scratch_types=[
            pltpu.VMEM((CHUNKS_PER_W // 2, CHUNK), jnp.int32),
            pltpu.VMEM((CHUNKS_PER_W // 2, CHUNK), jnp.int32),
            pltpu.VMEM((CHUNK, D), jnp.float32),
            pltpu.VMEM((CHUNK, D), jnp.float32),
            pltpu.VMEM_SHARED((ACC_ROWS, D), jnp.float32),
            pltpu.SemaphoreType.DMA,
            pltpu.SemaphoreType.DMA,
            pltpu.SemaphoreType.DMA,
            pltpu.SemaphoreType.DMA,
        ],
    )(_sc_scatter_body)


def _sc_scatter_body(table_h, idx_h, out_h, sidx, didx, rows0, rows1,
                     acc, g0, g1, s0, s1):
    c = lax.axis_index("c")
    s = lax.axis_index("s")
    wid = s * NC + c
    half = CHUNKS_PER_W // 2
    nt = half // 2
    HC = CHUNK // 2

    def gather2(j, rows, sem):
        pltpu.async_copy(
            table_h.at[sidx.at[j, pl.ds(0, HC)]], rows.at[pl.ds(0, HC)], sem)
        pltpu.async_copy(
            table_h.at[sidx.at[j, pl.ds(HC, HC)]], rows.at[pl.ds(HC, HC)],
            sem)

    def gwait2(j, rows, sem):
        pltpu.make_async_copy(
            table_h.at[sidx.at[j, pl.ds(0, HC)]], rows.at[pl.ds(0, HC)],
            sem).wait()
        pltpu.make_async_copy(
            table_h.at[sidx.at[j, pl.ds(HC, HC)]], rows.at[pl.ds(HC, HC)],
            sem).wait()

    pltpu.sync_copy(idx_h.at[0, wid, pl.ds(0, half)], sidx)
    pltpu.sync_copy(idx_h.at[1, wid, pl.ds(0, half)], didx)
    gather2(0, rows0, g0)
    _zero_vmem_2d(rows1, CHUNK, D)

    def zeroacc(k, _):
        pltpu.sync_copy(
            rows1, acc.at[pl.ds(s * ROWS_PER_TILE + k * CHUNK, CHUNK)])
        return 0
    lax.fori_loop(0, ROWS_PER_TILE // CHUNK, zeroacc, 0)
    plsc.subcore_barrier()

    for phase in range(2):
        if phase:
            pltpu.sync_copy(idx_h.at[0, wid, pl.ds(phase * half, half)], sidx)
            pltpu.sync_copy(idx_h.at[1, wid, pl.ds(phase * half, half)], didx)
            gather2(0, rows0, g0)

        def body(t, _):
            j0 = 2 * t
            gwait2(j0, rows0, g0)
            pltpu.async_copy(rows0, acc.at[didx.at[j0]], s0, add=True)

            @pl.when(t > 0)
            def _():
                pltpu.make_async_copy(
                    rows1, acc.at[didx.at[j0 - 1]], s1).wait()
            gather2(j0 + 1, rows1, g1)
            gwait2(j0 + 1, rows1, g1)
            pltpu.async_copy(rows1, acc.at[didx.at[j0 + 1]], s1, add=True)
            pltpu.make_async_copy(rows0, acc.at[didx.at[j0]], s0).wait()

            @pl.when(t < nt - 1)
            def _():
                gather2(j0 + 2, rows0, g0)
            return 0
        lax.fori_loop(0, nt, body, 0)
        pltpu.make_async_copy(rows1, acc.at[didx.at[half - 1]], s1).wait()
    plsc.subcore_barrier()
    pltpu.sync_copy(acc.at[pl.ds(s * ROWS_PER_TILE, ROWS_PER_TILE)],
                    out_h.at[c, pl.ds(s * ROWS_PER_TILE, ROWS_PER_TILE)])


_BM = 1264


def _project0(x, W1, degp3):
    def body(x_ref, w_ref, d0_ref, d1_ref, h_ref, dinv_ref):
        i = pl.program_id(0)
        deg = d0_ref[0] + d1_ref[0] + 1.0
        row = lax.broadcasted_iota(jnp.int32, (_BM, 1), 0) + i * _BM
        valid = row < N
        dinv = jnp.where(valid, lax.rsqrt(deg), 0.0)
        h = jnp.dot(x_ref[...], w_ref[...], preferred_element_type=jnp.float32)
        h_ref[...] = jnp.where(valid, h * dinv, 0.0)
        dinv_ref[...] = dinv
    return pl.pallas_call(
        body,
        grid=(NPAD // _BM,),
        in_specs=[
            pl.BlockSpec((_BM, D), lambda i: (i, 0)),
            pl.BlockSpec((D, D), lambda i: (0, 0)),
            pl.BlockSpec((1, _BM, 1), lambda i: (0, i, 0)),
            pl.BlockSpec((1, _BM, 1), lambda i: (1, i, 0)),
        ],
        out_specs=[
            pl.BlockSpec((_BM, D), lambda i: (i, 0)),
            pl.BlockSpec((_BM, 1), lambda i: (i, 0)),
        ],
        out_shape=[
            jax.ShapeDtypeStruct((NPAD, D), jnp.float32),
            jax.ShapeDtypeStruct((NPAD, 1), jnp.float32),
        ],
    )(x, W1, degp3, degp3)


def _combine_project(accp, hprev, dinv, b, Wn):
    def body(p0_ref, p1_ref, h_ref, dinv_ref, b_ref, w_ref, o_ref):
        a = p0_ref[0] + p1_ref[0]
        z = dinv_ref[...] * (a + h_ref[...]) + b_ref[...]
        y = jnp.maximum(z, 0.0)
        o_ref[...] = jnp.dot(
            y, w_ref[...], preferred_element_type=jnp.float32) * dinv_ref[...]
    return pl.pallas_call(
        body,
        grid=(NPAD // _BM,),
        in_specs=[
            pl.BlockSpec((1, _BM, D), lambda i: (0, i, 0)),
            pl.BlockSpec((1, _BM, D), lambda i: (1, i, 0)),
            pl.BlockSpec((_BM, D), lambda i: (i, 0)),
            pl.BlockSpec((_BM, 1), lambda i: (i, 0)),
            pl.BlockSpec((1, D), lambda i: (0, 0)),
            pl.BlockSpec((D, D), lambda i: (0, 0)),
        ],
        out_specs=pl.BlockSpec((_BM, D), lambda i: (i, 0)),
        out_shape=jax.ShapeDtypeStruct((NPAD, D), jnp.float32),
    )(accp, accp, hprev, dinv, b.reshape(1, D), Wn)


def _combine_pool_fc(accp, hprev, dinv, b, batch2, fcW, fcb):
    nsteps = NPAD // CHUNK

    def body(p0_ref, p1_ref, h_ref, dinv_ref, b_ref, bt_ref, w_ref, fb_ref,
             o_ref, pool_s, cnt_s):
        i = pl.program_id(0)

        @pl.when(i == 0)
        def _():
            pool_s[...] = jnp.zeros_like(pool_s)
            cnt_s[...] = jnp.zeros_like(cnt_s)

        a = p0_ref[0] + p1_ref[0]
        z = dinv_ref[...] * (a + h_ref[...]) + b_ref[...]
        y = jnp.maximum(z, 0.0)
        gids = lax.broadcasted_iota(jnp.int32, (NG, CHUNK), 0)
        maskf = jnp.where(gids == bt_ref[0], 1.0, 0.0)
        pool_s[...] += jnp.dot(maskf, y, preferred_element_type=jnp.float32)
        cnt_s[...] += jnp.sum(maskf, axis=1, keepdims=True)

        @pl.when(i == nsteps - 1)
        def _():
            g = pool_s[...] / jnp.maximum(cnt_s[...], 1.0)
            o = jnp.dot(g, w_ref[...], preferred_element_type=jnp.float32)
            o_ref[...] = jax.nn.sigmoid(o + fb_ref[...])

    return pl.pallas_call(
        body,
        grid=(nsteps,),
        in_specs=[
            pl.BlockSpec((1, CHUNK, D), lambda i: (0, i, 0)),
            pl.BlockSpec((1, CHUNK, D), lambda i: (1, i, 0)),
            pl.BlockSpec((CHUNK, D), lambda i: (i, 0)),
            pl.BlockSpec((CHUNK, 1), lambda i: (i, 0)),
            pl.BlockSpec((1, D), lambda i: (0, 0)),
            pl.BlockSpec((1, 1, CHUNK), lambda i: (i, 0, 0)),
            pl.BlockSpec((D, 1), lambda i: (0, 0)),
            pl.BlockSpec((1, 1), lambda i: (0, 0)),
        ],
        out_specs=pl.BlockSpec((NG, 1), lambda i: (0, 0)),
        out_shape=jax.ShapeDtypeStruct((NG, 1), jnp.float32),
        scratch_shapes=[
            pltpu.VMEM((NG, D), jnp.float32),
            pltpu.VMEM((NG, 1), jnp.float32),
        ],
    )(accp, accp, hprev, dinv, b.reshape(1, D), batch2, fcW,
      fcb.reshape(1, 1))


def kernel(x, edge_index, batch, W1, b1, W2, b2, W3, b3, fcW, fcb):
    ei = edge_index.astype(jnp.int32)
    pad_idx = N + (jnp.arange(E_PAD - E, dtype=jnp.int32) % (NPAD - N))
    pad2 = jnp.broadcast_to(pad_idx, (2, E_PAD - E))
    idx4 = jnp.concatenate([ei, pad2], axis=1).reshape(
        2, NW, CHUNKS_PER_W, CHUNK)

    degp = _make_sc_degree()(idx4)

    h1, dinv = _project0(x, W1, degp.reshape(NC, ACC_ROWS, 1))

    acc1 = _make_sc_scatter()(h1, idx4)
    h2 = _combine_project(acc1, h1, dinv, b1, W2)
    acc2 = _make_sc_scatter()(h2, idx4)
    h3 = _combine_project(acc2, h2, dinv, b2, W3)
    acc3 = _make_sc_scatter()(h3, idx4)

    batch2 = jnp.pad(batch.astype(jnp.int32), (0, NPAD - N),
                     constant_values=NG).reshape(NPAD // CHUNK, 1, CHUNK)
    return _combine_pool_fc(acc3, h3, dinv, b3, batch2, fcW, fcb)


# speedup vs baseline: 1.0354x; 1.0043x over previous
import functools

import jax
import jax.numpy as jnp
from jax import lax
from jax.experimental import pallas as pl
from jax.experimental.pallas import tpu as pltpu
from jax.experimental.pallas import tpu_sc as plsc

N = 10000
D = 128
NG = 64
E = 320000

NPAD = 10112
ROWS_PER_TILE = 640
ACC_ROWS = 10240
CHUNK = 128
NC = 2
NS = 16
NW = NC * NS
CHUNKS_PER_W = 80
E_PAD = NW * CHUNKS_PER_W * CHUNK

def _zero_vmem_2d(ref, rows, cols):
    def body(r, _):
        for k in range(cols // 16):
            ref[r, pl.ds(k * 16, 16)] = jnp.zeros((16,), jnp.float32)
        return 0
    lax.fori_loop(0, rows, body, 0)


def _zero_vmem_1d(ref, n):
    def body(j, _):
        ref[pl.ds(j * 16, 16)] = jnp.zeros((16,), jnp.float32)
        return 0
    lax.fori_loop(0, n // 16, body, 0)


NCH = E // CHUNK
CPW_DEG = 80
TAIL_DEG = NCH - 31 * CPW_DEG


@functools.cache
def _make_sc_degree():
    return functools.partial(
        pl.kernel,
        out_type=jax.ShapeDtypeStruct((NC, ACC_ROWS), jnp.float32),
        mesh=plsc.VectorSubcoreMesh(core_axis_name="c", subcore_axis_name="s"),
        scratch_types=[
            pltpu.VMEM((CPW_DEG, CHUNK), jnp.int32),
            pltpu.VMEM((CHUNK,), jnp.float32),
            pltpu.VMEM((ROWS_PER_TILE,), jnp.float32),
            pltpu.VMEM_SHARED((ACC_ROWS,), jnp.float32),
            pltpu.SemaphoreType.DMA,
        ],
    )(_sc_degree_body)


def _sc_degree_body(ei_h, out_h, didx_all, ones_v, zbuf, acc, sem):
    c = lax.axis_index("c")
    s = lax.axis_index("s")
    wid = s * NC + c
    ntr = jnp.where(wid < NW - 1, CPW_DEG, TAIL_DEG)

    @pl.when(wid < NW - 1)
    def _():
        pltpu.sync_copy(ei_h.at[1, pl.ds(wid * CPW_DEG, CPW_DEG)], didx_all)

    @pl.when(wid == NW - 1)
    def _():
        pltpu.sync_copy(ei_h.at[1, pl.ds((NW - 1) * CPW_DEG, TAIL_DEG)],
                        didx_all.at[pl.ds(0, TAIL_DEG)])

    def setone(j, _):
        ones_v[pl.ds(j * 16, 16)] = jnp.ones((16,), jnp.float32)
        return 0
    lax.fori_loop(0, CHUNK // 16, setone, 0)
    _zero_vmem_1d(zbuf, ROWS_PER_TILE)
    pltpu.sync_copy(zbuf, acc.at[pl.ds(s * ROWS_PER_TILE, ROWS_PER_TILE)])
    plsc.subcore_barrier()

    W = 8

    def fire(j, _):
        pltpu.async_copy(ones_v, acc.at[didx_all.at[j]], sem, add=True)

        @pl.when(j >= W)
        def _():
            pltpu.make_async_copy(
                ones_v, acc.at[didx_all.at[j - W]], sem).wait()
        return 0
    lax.fori_loop(0, ntr, fire, 0)

    def drain(j, _):
        pltpu.make_async_copy(
            ones_v, acc.at[didx_all.at[ntr - W + j]], sem).wait()
        return 0
    lax.fori_loop(0, W, drain, 0)
    plsc.subcore_barrier()
    pltpu.sync_copy(acc.at[pl.ds(s * ROWS_PER_TILE, ROWS_PER_TILE)],
                    out_h.at[c, pl.ds(s * ROWS_PER_TILE, ROWS_PER_TILE)])


@functools.cache
def _make_sc_scatter():
    return functools.partial(
        pl.kernel,
        out_type=jax.ShapeDtypeStruct((NC, ACC_ROWS, D), jnp.float32),
        mesh=plsc.VectorSubcoreMesh(core_axis_name="c", subcore_axis_name="s"),
        scratch_types=[
            pltpu.VMEM((CHUNKS_PER_W // 2, CHUNK), jnp.int32),
            pltpu.VMEM((CHUNKS_PER_W // 2, CHUNK), jnp.int32),
            pltpu.VMEM((CHUNK, D), jnp.float32),
            pltpu.VMEM((CHUNK, D), jnp.float32),
            pltpu.VMEM_SHARED((ACC_ROWS, D), jnp.float32),
            pltpu.SemaphoreType.DMA,
            pltpu.SemaphoreType.DMA,
            pltpu.SemaphoreType.DMA,
            pltpu.SemaphoreType.DMA,
        ],
    )(_sc_scatter_body)


def _sc_scatter_body(table_h, idx_h, out_h, sidx, didx, rows0, rows1,
                     acc, g0, g1, s0, s1):
    c = lax.axis_index("c")
    s = lax.axis_index("s")
    wid = s * NC + c
    half = CHUNKS_PER_W // 2
    nt = half // 2
    HC = CHUNK // 2

    def gather2(j, rows, sem):
        pltpu.async_copy(
            table_h.at[sidx.at[j, pl.ds(0, HC)]], rows.at[pl.ds(0, HC)], sem)
        pltpu.async_copy(
            table_h.at[sidx.at[j, pl.ds(HC, HC)]], rows.at[pl.ds(HC, HC)],
            sem)

    def gwait2(j, rows, sem):
        pltpu.make_async_copy(
            table_h.at[sidx.at[j, pl.ds(0, HC)]], rows.at[pl.ds(0, HC)],
            sem).wait()
        pltpu.make_async_copy(
            table_h.at[sidx.at[j, pl.ds(HC, HC)]], rows.at[pl.ds(HC, HC)],
            sem).wait()

    pltpu.sync_copy(idx_h.at[0, wid, pl.ds(0, half)], sidx)
    pltpu.sync_copy(idx_h.at[1, wid, pl.ds(0, half)], didx)
    gather2(0, rows0, g0)
    _zero_vmem_2d(rows1, CHUNK, D)

    def zeroacc(k, _):
        pltpu.sync_copy(
            rows1, acc.at[pl.ds(s * ROWS_PER_TILE + k * CHUNK, CHUNK)])
        return 0
    lax.fori_loop(0, ROWS_PER_TILE // CHUNK, zeroacc, 0)
    plsc.subcore_barrier()

    for phase in range(2):
        if phase:
            pltpu.sync_copy(idx_h.at[0, wid, pl.ds(phase * half, half)], sidx)
            pltpu.sync_copy(idx_h.at[1, wid, pl.ds(phase * half, half)], didx)
            gather2(0, rows0, g0)

        def body(t, _):
            j0 = 2 * t
            gwait2(j0, rows0, g0)
            pltpu.async_copy(rows0, acc.at[didx.at[j0]], s0, add=True)

            @pl.when(t > 0)
            def _():
                pltpu.make_async_copy(
                    rows1, acc.at[didx.at[j0 - 1]], s1).wait()
            gather2(j0 + 1, rows1, g1)
            gwait2(j0 + 1, rows1, g1)
            pltpu.async_copy(rows1, acc.at[didx.at[j0 + 1]], s1, add=True)
            pltpu.make_async_copy(rows0, acc.at[didx.at[j0]], s0).wait()

            @pl.when(t < nt - 1)
            def _():
                gather2(j0 + 2, rows0, g0)
            return 0
        lax.fori_loop(0, nt, body, 0)
        pltpu.make_async_copy(rows1, acc.at[didx.at[half - 1]], s1).wait()
    plsc.subcore_barrier()
    pltpu.sync_copy(acc.at[pl.ds(s * ROWS_PER_TILE, ROWS_PER_TILE)],
                    out_h.at[c, pl.ds(s * ROWS_PER_TILE, ROWS_PER_TILE)])


_BM = 1264


def _project0(x, W1, degp3):
    def body(x_ref, w_ref, d0_ref, d1_ref, h_ref, dinv_ref):
        i = pl.program_id(0)
        deg = d0_ref[0] + d1_ref[0] + 1.0
        row = lax.broadcasted_iota(jnp.int32, (_BM, 1), 0) + i * _BM
        valid = row < N
        dinv = jnp.where(valid, lax.rsqrt(deg), 0.0)
        h = jnp.dot(x_ref[...], w_ref[...], preferred_element_type=jnp.float32)
        h_ref[...] = jnp.where(valid, h * dinv, 0.0)
        dinv_ref[...] = dinv
    return pl.pallas_call(
        body,
        grid=(NPAD // _BM,),
        in_specs=[
            pl.BlockSpec((_BM, D), lambda i: (i, 0)),
            pl.BlockSpec((D, D), lambda i: (0, 0)),
            pl.BlockSpec((1, _BM, 1), lambda i: (0, i, 0)),
            pl.BlockSpec((1, _BM, 1), lambda i: (1, i, 0)),
        ],
        out_specs=[
            pl.BlockSpec((_BM, D), lambda i: (i, 0)),
            pl.BlockSpec((_BM, 1), lambda i: (i, 0)),
        ],
        out_shape=[
            jax.ShapeDtypeStruct((NPAD, D), jnp.float32),
            jax.ShapeDtypeStruct((NPAD, 1), jnp.float32),
        ],
    )(x, W1, degp3, degp3)


def _combine_project(accp, hprev, dinv, b, Wn):
    def body(p0_ref, p1_ref, h_ref, dinv_ref, b_ref, w_ref, o_ref):
        a = p0_ref[0] + p1_ref[0]
        z = dinv_ref[...] * (a + h_ref[...]) + b_ref[...]
        y = jnp.maximum(z, 0.0)
        o_ref[...] = jnp.dot(
            y, w_ref[...], preferred_element_type=jnp.float32) * dinv_ref[...]
    return pl.pallas_call(
        body,
        grid=(NPAD // _BM,),
        in_specs=[
            pl.BlockSpec((1, _BM, D), lambda i: (0, i, 0)),
            pl.BlockSpec((1, _BM, D), lambda i: (1, i, 0)),
            pl.BlockSpec((_BM, D), lambda i: (i, 0)),
            pl.BlockSpec((_BM, 1), lambda i: (i, 0)),
            pl.BlockSpec((1, D), lambda i: (0, 0)),
            pl.BlockSpec((D, D), lambda i: (0, 0)),
        ],
        out_specs=pl.BlockSpec((_BM, D), lambda i: (i, 0)),
        out_shape=jax.ShapeDtypeStruct((NPAD, D), jnp.float32),
    )(accp, accp, hprev, dinv, b.reshape(1, D), Wn)


def _combine_pool_fc(accp, hprev, dinv, b, batch2, fcW, fcb):
    nsteps = NPAD // CHUNK

    def body(p0_ref, p1_ref, h_ref, dinv_ref, b_ref, bt_ref, w_ref, fb_ref,
             o_ref, pool_s, cnt_s):
        i = pl.program_id(0)

        @pl.when(i == 0)
        def _():
            pool_s[...] = jnp.zeros_like(pool_s)
            cnt_s[...] = jnp.zeros_like(cnt_s)

        a = p0_ref[0] + p1_ref[0]
        z = dinv_ref[...] * (a + h_ref[...]) + b_ref[...]
        y = jnp.maximum(z, 0.0)
        gids = lax.broadcasted_iota(jnp.int32, (NG, CHUNK), 0)
        maskf = jnp.where(gids == bt_ref[0], 1.0, 0.0)
        pool_s[...] += jnp.dot(maskf, y, preferred_element_type=jnp.float32)
        cnt_s[...] += jnp.sum(maskf, axis=1, keepdims=True)

        @pl.when(i == nsteps - 1)
        def _():
            g = pool_s[...] / jnp.maximum(cnt_s[...], 1.0)
            o = jnp.dot(g, w_ref[...], preferred_element_type=jnp.float32)
            o_ref[...] = jax.nn.sigmoid(o + fb_ref[...])

    return pl.pallas_call(
        body,
        grid=(nsteps,),
        in_specs=[
            pl.BlockSpec((1, CHUNK, D), lambda i: (0, i, 0)),
            pl.BlockSpec((1, CHUNK, D), lambda i: (1, i, 0)),
            pl.BlockSpec((CHUNK, D), lambda i: (i, 0)),
            pl.BlockSpec((CHUNK, 1), lambda i: (i, 0)),
            pl.BlockSpec((1, D), lambda i: (0, 0)),
            pl.BlockSpec((1, 1, CHUNK), lambda i: (i, 0, 0)),
            pl.BlockSpec((D, 1), lambda i: (0, 0)),
            pl.BlockSpec((1, 1), lambda i: (0, 0)),
        ],
        out_specs=pl.BlockSpec((NG, 1), lambda i: (0, 0)),
        out_shape=jax.ShapeDtypeStruct((NG, 1), jnp.float32),
        scratch_shapes=[
            pltpu.VMEM((NG, D), jnp.float32),
            pltpu.VMEM((NG, 1), jnp.float32),
        ],
    )(accp, accp, hprev, dinv, b.reshape(1, D), batch2, fcW,
      fcb.reshape(1, 1))


def kernel(x, edge_index, batch, W1, b1, W2, b2, W3, b3, fcW, fcb):
    ei = edge_index.astype(jnp.int32)
    pad_idx = N + (jnp.arange(E_PAD - E, dtype=jnp.int32) % (NPAD - N))
    pad2 = jnp.broadcast_to(pad_idx, (2, E_PAD - E))
    idx4 = jnp.concatenate([ei, pad2], axis=1).reshape(
        2, NW, CHUNKS_PER_W, CHUNK)

    degp = _make_sc_degree()(ei.reshape(2, NCH, CHUNK))

    h1, dinv = _project0(x, W1, degp.reshape(NC, ACC_ROWS, 1))

    acc1 = _make_sc_scatter()(h1, idx4)
    h2 = _combine_project(acc1, h1, dinv, b1, W2)
    acc2 = _make_sc_scatter()(h2, idx4)
    h3 = _combine_project(acc2, h2, dinv, b2, W3)
    acc3 = _make_sc_scatter()(h3, idx4)

    batch2 = jnp.pad(batch.astype(jnp.int32), (0, NPAD - N),
                     constant_values=NG).reshape(NPAD // CHUNK, 1, CHUNK)
    return _combine_pool_fc(acc3, h3, dinv, b3, batch2, fcW, fcb)
